# Initial kernel scaffold; baseline (speedup 1.0000x reference)
import jax
import jax.numpy as jnp
from jax.experimental import pallas as pl


def kernel(latent_vec, params, batch):
    raise NotImplementedError("write your pallas kernel here")


# speedup vs baseline: 92.5461x; 92.5461x over previous
import jax
import jax.numpy as jnp
from jax import lax
from jax.experimental import pallas as pl
from jax.experimental.pallas import tpu as pltpu


_DBLK = 128
_SC = 16
_RMAX = 512

_LAYER_DIMS = [(20, 16), (16, 4)] * 4


def _dgT(lhs, rhs, prec=None):
    return lax.dot_general(lhs, rhs, (((0,), (0,)), ((), ())),
                           precision=prec, preferred_element_type=jnp.float32)


def _rb(t):
    return t.astype(jnp.bfloat16).astype(jnp.float32)


def _forward_body(N_pad, NBLK, B,
                  latent_ref, batch_row_ref, gs_row_ref, ge_row_ref,
                  noiseT_ref, blk_lo_ref, blk_hi_ref, wref,
                  out0, out1, out2, out3, out_nn,
                  xT, anat, aT, bT, AT, BT, ST, Rdiag, Rsl, Mref,
                  rbuf, rhbm, sem):
    f32 = jnp.float32
    HI = lax.Precision.HIGHEST

    def dotD(p, q):
        return jnp.dot(p, q, preferred_element_type=f32)

    wi = {'i': 0}

    def nxt():
        r = wref[wi['i']]
        wi['i'] += 1
        return r[...]

    latent = latent_ref[...]
    W1, b1, W2, b2, W3, b3 = (nxt() for _ in range(6))
    h = dotD(latent, W1) + b1
    h = jnp.where(h > 0, h, 0.01 * h)
    h = dotD(h, W2) + b2
    h = jnp.where(h > 0, h, 0.01 * h)
    nn_head = dotD(h, W3) + b3
    out_nn[...] = jnp.broadcast_to(nn_head, (B, 128))

    ltW1, ltW2 = nxt(), nxt()
    e = dotD(latent, ltW1)
    e = jnp.where(e > 0, e, 0.01 * e)
    emb = dotD(e, ltW2)

    brow = batch_row_ref[...]
    gi_row = lax.broadcasted_iota(jnp.int32, (B, N_pad), 0)
    Mref[...] = (gi_row == brow).astype(f32)
    counts = jnp.sum(Mref[...], axis=1, keepdims=True)
    cinv = 1.0 / jnp.maximum(counts, 1.0)

    xT[0:16, :] = _dgT(emb, Mref[...], HI)
    xT[16:20, :] = noiseT_ref[...]
    z = jnp.zeros((1, N_pad), f32)
    one = jnp.ones((1, N_pad), f32)
    Rdiag[...] = jnp.concatenate([one, z, z, z, z], 0)
    Rsl[...] = jnp.concatenate([z, one, z, z, z], 0)

    for li in range(8):
        idim, o = _LAYER_DIMS[li]
        (We_sT, We_dT, We_s, be_col, We_aT, Wa_sT, Wa_dT, Wa_s, Wa_aT,
         ba_col, Wn_xT, Wn_aT, bn_col) = (nxt() for _ in range(13))

        if li >= 2 and li % 2 == 0:
            xT[4:20, :] = _dgT(emb, Mref[...], HI)

        x_t = xT[0:idim, :]
        aT[0:o, :] = dotD(We_sT, x_t)
        bT[0:o, :] = dotD(We_dT, x_t) + be_col
        AT[...] = dotD(Wa_sT, x_t)
        BT[...] = dotD(Wa_dT, x_t)
        anat[:, 0:o] = _dgT(x_t, We_s)
        anat[:, 16:21] = _dgT(x_t, Wa_s)

        def dblk(db, _, o=o, li=li, We_aT=We_aT, Wa_aT=Wa_aT, ba_col=ba_col):
            d0 = db * _DBLK
            if li > 0:
                cp = pltpu.make_async_copy(rhbm.at[db], rbuf, sem)
                cp.start()
                cp.wait()
            else:
                rbuf[:, 0:_DBLK] = jnp.ones((_RMAX, _DBLK), jnp.bfloat16)
                rbuf[:, _DBLK:] = jnp.zeros((_RMAX, 4 * _DBLK), jnp.bfloat16)
            bt = bT[0:o, pl.ds(d0, _DBLK)]
            Bt = BT[0:5, pl.ds(d0, _DBLK)]
            gs_d = gs_row_ref[:, pl.ds(d0, _DBLK)]
            ge_d = ge_row_ref[:, pl.ds(d0, _DBLK)]
            lo = (blk_lo_ref[db] // _SC) * _SC
            hi = blk_hi_ref[db]
            nit = jnp.minimum((hi - lo + _SC - 1) // _SC, _RMAX // _SC)
            si0 = lax.broadcasted_iota(jnp.int32, (_SC, 1), 0)

            def sstep(it, accs, lo=lo, o=o, bt=bt, Bt=Bt,
                      gs_d=gs_d, ge_d=ge_d, si0=si0,
                      We_aT=We_aT, Wa_aT=Wa_aT, ba_col=ba_col):
                r0 = it * _SC
                s0 = lo + r0
                au = anat[pl.ds(s0, _SC), 0:21]
                si = si0 + s0
                mask = (si >= gs_d) & (si < ge_d)
                rt = [rbuf[pl.ds(r0, _SC),
                           k * _DBLK:(k + 1) * _DBLK].astype(f32)
                      for k in range(5)]
                out = []
                for f in range(o):
                    t = au[:, f:f + 1] + bt[f:f + 1, :]
                    for k in range(5):
                        t = t + rt[k] * We_aT[f:f + 1, k:k + 1]
                    t = jnp.maximum(t, 0.0)
                    out.append(accs[f] + jnp.where(mask, t, 0.0))
                for k in range(5):
                    nr = au[:, 16 + k:17 + k] + Bt[k:k + 1, :]
                    for j in range(5):
                        nr = nr + rt[j] * Wa_aT[k:k + 1, j:j + 1]
                    nr = nr + ba_col[k:k + 1, :]
                    rbuf[pl.ds(r0, _SC), k * _DBLK:(k + 1) * _DBLK] = (
                        nr.astype(jnp.bfloat16))
                return tuple(out)

            accs = lax.fori_loop(
                0, nit, sstep,
                tuple(jnp.zeros((_SC, _DBLK), f32) for _ in range(o)))
            st = jnp.concatenate(
                [jnp.sum(accs[f], axis=0, keepdims=True) for f in range(o)], 0)
            ST[0:o, pl.ds(d0, _DBLK)] = st
            if li < 7:
                cp2 = pltpu.make_async_copy(rbuf, rhbm.at[db], sem)
                cp2.start()
                cp2.wait()
            return 0

        lax.fori_loop(0, NBLK, dblk, 0)

        abT = aT[0:o, :] + bT[0:o, :]
        t_ns = abT
        t_sl = abT
        for k in range(5):
            wk = We_aT[:, k:k + 1]
            t_ns = t_ns + Rdiag[k:k + 1, :] * wk
            t_sl = t_sl + Rsl[k:k + 1, :] * wk
        aggT = (ST[0:o, :] - jnp.maximum(t_ns, 0.0)
                + jnp.maximum(t_sl, 0.0))
        xnewT = dotD(Wn_xT, x_t) + dotD(Wn_aT, aggT) + bn_col

        if li < 7:
            ABt = AT[...] + BT[...]
            nd = []
            ns = []
            for k in range(5):
                td = ABt[k:k + 1, :]
                ts = ABt[k:k + 1, :]
                for j in range(5):
                    wkj = Wa_aT[k:k + 1, j:j + 1]
                    td = td + Rdiag[j:j + 1, :] * wkj
                    ts = ts + Rsl[j:j + 1, :] * wkj
                nd.append(td + ba_col[k:k + 1, :])
                ns.append(ts + ba_col[k:k + 1, :])
            Rdiag[...] = _rb(jnp.concatenate(nd, 0))
            Rsl[...] = _rb(jnp.concatenate(ns, 0))

            mean = lax.dot_general(
                Mref[...], xnewT, (((1,), (1,)), ((), ())),
                precision=HI, preferred_element_type=f32) * cinv
            xcT = xnewT - _dgT(mean, Mref[...], HI)
            sqT = jnp.sum(xcT * xcT, axis=0, keepdims=True)
            msq = lax.dot_general(
                Mref[...], sqT, (((1,), (1,)), ((), ())),
                precision=HI, preferred_element_type=f32) * cinv
            msq_n = _dgT(msq, Mref[...], HI)
            xnewT = jnp.maximum(xcT / jnp.sqrt(msq_n + 1e-6), 0.0)
        xT[0:o, :] = xnewT
        if li % 2 == 1:
            outref = (out0, out1, out2, out3)[li // 2]
            outref[...] = xnewT


def _pack_weights(params):
    ws = []
    p = params['nnl']
    ws += [p['W1'], p['b1'].reshape(1, -1), p['W2'], p['b2'].reshape(1, -1),
           p['W3'], p['b3'].reshape(1, -1)]
    ws += [params['lt']['W1'], params['lt']['W2']]
    for name in ['gnn1', 'gnn2', 'gnn3', 'gnn4']:
        for li in range(2):
            pp = params[name][li]
            i = pp['Wn'].shape[0] - pp['We'].shape[1]
            o = pp['We'].shape[1]
            We, Wa, Wn = pp['We'], pp['Wa'], pp['Wn']
            We_s, We_d, We_a = We[:i], We[i:2 * i], We[2 * i:]
            ws += [We_s.T, We_d.T, We_s, pp['be'].reshape(o, 1),
                   _rb(We_a.T),
                   Wa[:i].T, Wa[i:2 * i].T, Wa[:i],
                   _rb(Wa[2 * i:].T),
                   pp['ba'].reshape(5, 1),
                   Wn[:i].T, Wn[i:].T, pp['bn'].reshape(o, 1)]
    return ws


@jax.jit
def kernel(latent_vec, params, batch):
    B = latent_vec.shape[0]
    N = batch.shape[0]
    batch = batch.astype(jnp.int32)
    NBLK = -(-N // _DBLK)
    N_pad = NBLK * _DBLK

    nn = jnp.bincount(batch, length=B)
    ends = jnp.cumsum(nn).astype(jnp.int32)
    starts = ends - nn
    gs_node = starts[batch]
    ge_node = ends[batch]
    d_first = jnp.minimum(jnp.arange(NBLK, dtype=jnp.int32) * _DBLK, N - 1)
    d_last = jnp.minimum(d_first + _DBLK - 1, N - 1)
    blk_lo = gs_node[d_first].astype(jnp.int32)
    blk_hi = ge_node[d_last].astype(jnp.int32)

    pad = N_pad - N
    batch_row = jnp.pad(batch, (0, pad), constant_values=B).reshape(1, N_pad)
    gs_row = jnp.pad(gs_node, (0, pad)).reshape(1, N_pad).astype(jnp.int32)
    ge_row = jnp.pad(ge_node, (0, pad)).reshape(1, N_pad).astype(jnp.int32)
    noise = jax.random.uniform(jax.random.key(1), (N, 4), jnp.float32)
    noiseT = jnp.pad(noise, ((0, pad), (0, 0))).T

    ws = _pack_weights(params)
    nw = len(ws)

    vspec = pl.BlockSpec(memory_space=pltpu.VMEM)
    sspec = pl.BlockSpec(memory_space=pltpu.SMEM)

    def body(latent_ref, batch_row_ref, gs_row_ref, ge_row_ref, noiseT_ref,
             blk_lo_ref, blk_hi_ref, *rest):
        wref = rest[:nw]
        (out0, out1, out2, out3, out_nn, rhbm,
         xT, anat, aT, bT, AT, BT, ST, Rdiag, Rsl, Mref,
         rbuf, sem) = rest[nw:]
        _forward_body(N_pad, NBLK, B,
                      latent_ref, batch_row_ref, gs_row_ref, ge_row_ref,
                      noiseT_ref, blk_lo_ref, blk_hi_ref, wref,
                      out0, out1, out2, out3, out_nn,
                      xT, anat, aT, bT, AT, BT, ST, Rdiag, Rsl, Mref,
                      rbuf, rhbm, sem)

    f32 = jnp.float32
    outs = pl.pallas_call(
        body,
        out_shape=[jax.ShapeDtypeStruct((4, N_pad), f32)] * 4
        + [jax.ShapeDtypeStruct((B, 128), f32),
           jax.ShapeDtypeStruct((NBLK, _RMAX, 5 * _DBLK), jnp.bfloat16)],
        in_specs=[vspec] * 5 + [sspec, sspec] + [vspec] * nw,
        out_specs=[vspec] * 5 + [pl.BlockSpec(memory_space=pl.ANY)],
        scratch_shapes=[
            pltpu.VMEM((20, N_pad), f32),
            pltpu.VMEM((N_pad, 21), f32),
            pltpu.VMEM((16, N_pad), f32),
            pltpu.VMEM((16, N_pad), f32),
            pltpu.VMEM((5, N_pad), f32),
            pltpu.VMEM((5, N_pad), f32),
            pltpu.VMEM((16, N_pad), f32),
            pltpu.VMEM((5, N_pad), f32),
            pltpu.VMEM((5, N_pad), f32),
            pltpu.VMEM((B, N_pad), f32),
            pltpu.VMEM((_RMAX, 5 * _DBLK), jnp.bfloat16),
            pltpu.SemaphoreType.DMA,
        ],
    )(latent_vec, batch_row, gs_row, ge_row, noiseT, blk_lo, blk_hi, *ws)

    o0, o1, o2, o3, onn = outs[:5]
    return (o0[:, :N].T, o1[:, :N].T, o2[:, :N].T, o3[:, :N].T, onn[:, 0])


# speedup vs baseline: 124.9756x; 1.3504x over previous
import jax
import jax.numpy as jnp
from jax import lax
from jax.experimental import pallas as pl
from jax.experimental.pallas import tpu as pltpu


_DBLK = 128
_SC = 16
_RMAX = 512

_LAYER_DIMS = [(20, 16), (16, 4)] * 4


def _dgT(lhs, rhs, prec=None):
    return lax.dot_general(lhs, rhs, (((0,), (0,)), ((), ())),
                           precision=prec, preferred_element_type=jnp.float32)


def _rb(t):
    return t.astype(jnp.bfloat16).astype(jnp.float32)


def _forward_body(N_pad, NBLK, B,
                  latent_ref, batch_row_ref, gs_row_ref, ge_row_ref,
                  noiseT_ref, blk_lo_ref, blk_hi_ref, wref,
                  out0, out1, out2, out3, out_nn,
                  xT, anat, aT, bT, AT, BT, ST, Rdiag, Rsl, Mref,
                  rbuf, rhbm, sem_in, sem_out):
    f32 = jnp.float32
    HI = lax.Precision.HIGHEST

    def dotD(p, q):
        return jnp.dot(p, q, preferred_element_type=f32)

    wi = {'i': 0}

    def nxt():
        r = wref[wi['i']]
        wi['i'] += 1
        return r[...]

    latent = latent_ref[...]
    W1, b1, W2, b2, W3, b3 = (nxt() for _ in range(6))
    h = dotD(latent, W1) + b1
    h = jnp.where(h > 0, h, 0.01 * h)
    h = dotD(h, W2) + b2
    h = jnp.where(h > 0, h, 0.01 * h)
    nn_head = dotD(h, W3) + b3
    out_nn[...] = jnp.broadcast_to(nn_head, (B, 128))

    ltW1, ltW2 = nxt(), nxt()
    e = dotD(latent, ltW1)
    e = jnp.where(e > 0, e, 0.01 * e)
    emb = dotD(e, ltW2)

    brow = batch_row_ref[...]
    gi_row = lax.broadcasted_iota(jnp.int32, (B, N_pad), 0)
    Mref[...] = (gi_row == brow).astype(f32)
    counts = jnp.sum(Mref[...], axis=1, keepdims=True)
    cinv = 1.0 / jnp.maximum(counts, 1.0)

    xT[0:16, :] = _dgT(emb, Mref[...], HI)
    xT[16:20, :] = noiseT_ref[...]
    z = jnp.zeros((1, N_pad), f32)
    one = jnp.ones((1, N_pad), f32)
    Rdiag[...] = jnp.concatenate([one, z, z, z, z], 0)
    Rsl[...] = jnp.concatenate([z, one, z, z, z], 0)

    for li in range(8):
        idim, o = _LAYER_DIMS[li]
        (We_sT, We_dT, We_s, be_col, We_aT, Wa_sT, Wa_dT, Wa_s, Wa_aT,
         ba_col, Wn_xT, Wn_aT, bn_col) = (nxt() for _ in range(13))

        if li >= 2 and li % 2 == 0:
            xT[4:20, :] = _dgT(emb, Mref[...], HI)

        x_t = xT[0:idim, :]
        aT[0:o, :] = dotD(We_sT, x_t)
        bT[0:o, :] = dotD(We_dT, x_t) + be_col
        AT[...] = dotD(Wa_sT, x_t)
        BT[...] = dotD(Wa_dT, x_t)
        anat[:, 0:o] = _dgT(x_t, We_s)
        anat[:, 16:21] = _dgT(x_t, Wa_s)

        def dblk(db, _, o=o, li=li, We_aT=We_aT, Wa_aT=Wa_aT, ba_col=ba_col):
            d0 = db * _DBLK
            cur = lax.rem(db, 2)
            if 0 < li < 7:
                @pl.when(db + 1 < NBLK)
                def _():
                    @pl.when(db >= 1)
                    def _():
                        pltpu.make_async_copy(
                            rbuf.at[1 - cur], rhbm.at[db - 1],
                            sem_out.at[1 - cur]).wait()
                    pltpu.make_async_copy(
                        rhbm.at[db + 1], rbuf.at[1 - cur],
                        sem_in.at[1 - cur]).start()
                pltpu.make_async_copy(
                    rhbm.at[db], rbuf.at[cur], sem_in.at[cur]).wait()
            elif li == 7:
                @pl.when(db + 1 < NBLK)
                def _():
                    pltpu.make_async_copy(
                        rhbm.at[db + 1], rbuf.at[1 - cur],
                        sem_in.at[1 - cur]).start()
                pltpu.make_async_copy(
                    rhbm.at[db], rbuf.at[cur], sem_in.at[cur]).wait()
            else:
                @pl.when(db >= 2)
                def _():
                    pltpu.make_async_copy(
                        rbuf.at[cur], rhbm.at[db - 2],
                        sem_out.at[cur]).wait()
            bt = bT[0:o, pl.ds(d0, _DBLK)]
            Bt = BT[0:5, pl.ds(d0, _DBLK)]
            gs_d = gs_row_ref[:, pl.ds(d0, _DBLK)]
            ge_d = ge_row_ref[:, pl.ds(d0, _DBLK)]
            lo = (blk_lo_ref[db] // _SC) * _SC
            hi = blk_hi_ref[db]
            nit = jnp.minimum((hi - lo + _SC - 1) // _SC, _RMAX // _SC)
            si0 = lax.broadcasted_iota(jnp.int32, (_SC, 1), 0)

            def sstep(it, accs, lo=lo, o=o, li=li, cur=cur, bt=bt, Bt=Bt,
                      gs_d=gs_d, ge_d=ge_d, si0=si0,
                      We_aT=We_aT, Wa_aT=Wa_aT, ba_col=ba_col):
                r0 = it * _SC
                s0 = lo + r0
                au = anat[pl.ds(s0, _SC), 0:21]
                si = si0 + s0
                maskf = ((si >= gs_d) & (si < ge_d)).astype(f32)
                if li > 0:
                    rt = [rbuf[cur, pl.ds(r0, _SC),
                               k * _DBLK:(k + 1) * _DBLK].astype(f32)
                          for k in range(5)]
                out = []
                for f in range(o):
                    t = au[:, f:f + 1] + bt[f:f + 1, :]
                    if li > 0:
                        for k in range(5):
                            t = t + rt[k] * We_aT[f:f + 1, k:k + 1]
                    else:
                        t = t + We_aT[f:f + 1, 0:1]
                    out.append(accs[f] + jnp.maximum(t, 0.0) * maskf)
                if li < 7:
                    for k in range(5):
                        nr = au[:, 16 + k:17 + k] + Bt[k:k + 1, :]
                        if li > 0:
                            for j in range(5):
                                nr = nr + rt[j] * Wa_aT[k:k + 1, j:j + 1]
                        else:
                            nr = nr + Wa_aT[k:k + 1, 0:1]
                        nr = nr + ba_col[k:k + 1, :]
                        rbuf[cur, pl.ds(r0, _SC),
                             k * _DBLK:(k + 1) * _DBLK] = (
                                 nr.astype(jnp.bfloat16))
                return tuple(out)

            accs = lax.fori_loop(
                0, nit, sstep,
                tuple(jnp.zeros((_SC, _DBLK), f32) for _ in range(o)))
            st = jnp.concatenate(
                [jnp.sum(accs[f], axis=0, keepdims=True) for f in range(o)], 0)
            ST[0:o, pl.ds(d0, _DBLK)] = st
            if li < 7:
                pltpu.make_async_copy(
                    rbuf.at[cur], rhbm.at[db], sem_out.at[cur]).start()
            return 0

        if li > 0:
            pltpu.make_async_copy(rhbm.at[0], rbuf.at[0], sem_in.at[0]).start()
        lax.fori_loop(0, NBLK, dblk, 0)
        if li < 7:
            pltpu.make_async_copy(
                rbuf.at[(NBLK - 1) % 2], rhbm.at[NBLK - 1],
                sem_out.at[(NBLK - 1) % 2]).wait()
            if NBLK >= 2:
                pltpu.make_async_copy(
                    rbuf.at[(NBLK - 2) % 2], rhbm.at[NBLK - 2],
                    sem_out.at[(NBLK - 2) % 2]).wait()

        abT = aT[0:o, :] + bT[0:o, :]
        t_ns = abT
        t_sl = abT
        for k in range(5):
            wk = We_aT[:, k:k + 1]
            t_ns = t_ns + Rdiag[k:k + 1, :] * wk
            t_sl = t_sl + Rsl[k:k + 1, :] * wk
        aggT = (ST[0:o, :] - jnp.maximum(t_ns, 0.0)
                + jnp.maximum(t_sl, 0.0))
        xnewT = dotD(Wn_xT, x_t) + dotD(Wn_aT, aggT) + bn_col

        if li < 7:
            ABt = AT[...] + BT[...]
            nd = []
            ns = []
            for k in range(5):
                td = ABt[k:k + 1, :]
                ts = ABt[k:k + 1, :]
                for j in range(5):
                    wkj = Wa_aT[k:k + 1, j:j + 1]
                    td = td + Rdiag[j:j + 1, :] * wkj
                    ts = ts + Rsl[j:j + 1, :] * wkj
                nd.append(td + ba_col[k:k + 1, :])
                ns.append(ts + ba_col[k:k + 1, :])
            Rdiag[...] = _rb(jnp.concatenate(nd, 0))
            Rsl[...] = _rb(jnp.concatenate(ns, 0))

            mean = lax.dot_general(
                Mref[...], xnewT, (((1,), (1,)), ((), ())),
                precision=HI, preferred_element_type=f32) * cinv
            xcT = xnewT - _dgT(mean, Mref[...], HI)
            sqT = jnp.sum(xcT * xcT, axis=0, keepdims=True)
            msq = lax.dot_general(
                Mref[...], sqT, (((1,), (1,)), ((), ())),
                precision=HI, preferred_element_type=f32) * cinv
            msq_n = _dgT(msq, Mref[...], HI)
            xnewT = jnp.maximum(xcT / jnp.sqrt(msq_n + 1e-6), 0.0)
        xT[0:o, :] = xnewT
        if li % 2 == 1:
            outref = (out0, out1, out2, out3)[li // 2]
            outref[...] = xnewT


def _pack_weights(params):
    ws = []
    p = params['nnl']
    ws += [p['W1'], p['b1'].reshape(1, -1), p['W2'], p['b2'].reshape(1, -1),
           p['W3'], p['b3'].reshape(1, -1)]
    ws += [params['lt']['W1'], params['lt']['W2']]
    for name in ['gnn1', 'gnn2', 'gnn3', 'gnn4']:
        for li in range(2):
            pp = params[name][li]
            i = pp['Wn'].shape[0] - pp['We'].shape[1]
            o = pp['We'].shape[1]
            We, Wa, Wn = pp['We'], pp['Wa'], pp['Wn']
            We_s, We_d, We_a = We[:i], We[i:2 * i], We[2 * i:]
            ws += [We_s.T, We_d.T, We_s, pp['be'].reshape(o, 1),
                   _rb(We_a.T),
                   Wa[:i].T, Wa[i:2 * i].T, Wa[:i],
                   _rb(Wa[2 * i:].T),
                   pp['ba'].reshape(5, 1),
                   Wn[:i].T, Wn[i:].T, pp['bn'].reshape(o, 1)]
    return ws


@jax.jit
def kernel(latent_vec, params, batch):
    B = latent_vec.shape[0]
    N = batch.shape[0]
    batch = batch.astype(jnp.int32)
    NBLK = -(-N // _DBLK)
    N_pad = NBLK * _DBLK

    nn = jnp.bincount(batch, length=B)
    ends = jnp.cumsum(nn).astype(jnp.int32)
    starts = ends - nn
    gs_node = starts[batch]
    ge_node = ends[batch]
    d_first = jnp.minimum(jnp.arange(NBLK, dtype=jnp.int32) * _DBLK, N - 1)
    d_last = jnp.minimum(d_first + _DBLK - 1, N - 1)
    blk_lo = gs_node[d_first].astype(jnp.int32)
    blk_hi = ge_node[d_last].astype(jnp.int32)

    pad = N_pad - N
    batch_row = jnp.pad(batch, (0, pad), constant_values=B).reshape(1, N_pad)
    gs_row = jnp.pad(gs_node, (0, pad)).reshape(1, N_pad).astype(jnp.int32)
    ge_row = jnp.pad(ge_node, (0, pad)).reshape(1, N_pad).astype(jnp.int32)
    noise = jax.random.uniform(jax.random.key(1), (N, 4), jnp.float32)
    noiseT = jnp.pad(noise, ((0, pad), (0, 0))).T

    ws = _pack_weights(params)
    nw = len(ws)

    vspec = pl.BlockSpec(memory_space=pltpu.VMEM)
    sspec = pl.BlockSpec(memory_space=pltpu.SMEM)

    def body(latent_ref, batch_row_ref, gs_row_ref, ge_row_ref, noiseT_ref,
             blk_lo_ref, blk_hi_ref, *rest):
        wref = rest[:nw]
        (out0, out1, out2, out3, out_nn, rhbm,
         xT, anat, aT, bT, AT, BT, ST, Rdiag, Rsl, Mref,
         rbuf, sem_in, sem_out) = rest[nw:]
        _forward_body(N_pad, NBLK, B,
                      latent_ref, batch_row_ref, gs_row_ref, ge_row_ref,
                      noiseT_ref, blk_lo_ref, blk_hi_ref, wref,
                      out0, out1, out2, out3, out_nn,
                      xT, anat, aT, bT, AT, BT, ST, Rdiag, Rsl, Mref,
                      rbuf, rhbm, sem_in, sem_out)

    f32 = jnp.float32
    outs = pl.pallas_call(
        body,
        out_shape=[jax.ShapeDtypeStruct((4, N_pad), f32)] * 4
        + [jax.ShapeDtypeStruct((B, 128), f32),
           jax.ShapeDtypeStruct((NBLK, _RMAX, 5 * _DBLK), jnp.bfloat16)],
        in_specs=[vspec] * 5 + [sspec, sspec] + [vspec] * nw,
        out_specs=[vspec] * 5 + [pl.BlockSpec(memory_space=pl.ANY)],
        scratch_shapes=[
            pltpu.VMEM((20, N_pad), f32),
            pltpu.VMEM((N_pad, 21), f32),
            pltpu.VMEM((16, N_pad), f32),
            pltpu.VMEM((16, N_pad), f32),
            pltpu.VMEM((5, N_pad), f32),
            pltpu.VMEM((5, N_pad), f32),
            pltpu.VMEM((16, N_pad), f32),
            pltpu.VMEM((5, N_pad), f32),
            pltpu.VMEM((5, N_pad), f32),
            pltpu.VMEM((B, N_pad), f32),
            pltpu.VMEM((2, _RMAX, 5 * _DBLK), jnp.bfloat16),
            pltpu.SemaphoreType.DMA((2,)),
            pltpu.SemaphoreType.DMA((2,)),
        ],
    )(latent_vec, batch_row, gs_row, ge_row, noiseT, blk_lo, blk_hi, *ws)

    o0, o1, o2, o3, onn = outs[:5]
    return (o0[:, :N].T, o1[:, :N].T, o2[:, :N].T, o3[:, :N].T, onn[:, 0])
